# Initial kernel scaffold; baseline (speedup 1.0000x reference)
import jax
import jax.numpy as jnp
from jax.experimental import pallas as pl


def kernel(x, shuf, adj_indices, adj_values, neighbor_indices, neighbor_values, temp):
    raise NotImplementedError("write your pallas kernel here")


# speedup vs baseline: 4.4612x; 4.4612x over previous
import functools

import jax
import jax.numpy as jnp
from jax import lax
from jax.experimental import pallas as pl
from jax.experimental.pallas import tpu as pltpu
from jax.experimental.pallas import tpu_sc as plsc

N = 10000
E = 320000
D = 128
G = 128
NC = 2
NS = 16
NGRP = 2528
GP_TEC = NGRP // (NC * NS)
EPAD = NGRP * G
RPS = N // NS


def _zero_rows(rows_v, n_rows, width):
    z16 = jnp.zeros((16,), jnp.float32)

    def body(i, _):
        for j in range(width // 16):
            rows_v[i, pl.ds(j * 16, 16)] = z16
        return 0

    lax.fori_loop(0, n_rows, body, 0)


def _zero_acc_slice(rows_v, acc, base, width):
    for k in range(4):
        pltpu.sync_copy(rows_v, acc.at[pl.ds(base + k * G, G)])
    pltpu.sync_copy(rows_v.at[pl.ds(0, RPS - 4 * G)],
                    acc.at[pl.ds(base + 4 * G, RPS - 4 * G)])


def _spmm_body(use_dis, col_hbm, row_hbm, val_hbm, x_hbm, dis_hbm, out_hbm,
               col_v, row_v, val_v, rows_v, dis_v, acc, sem):
    c = lax.axis_index("c")
    s = lax.axis_index("s")
    _zero_rows(rows_v, G, D)
    _zero_acc_slice(rows_v, acc, s * RPS, D)
    if use_dis:
        pltpu.sync_copy(dis_hbm, dis_v)
    plsc.subcore_barrier()

    g0 = (c * NS + s) * GP_TEC

    def grp_body(g, _):
        grp = g0 + g
        pltpu.sync_copy(col_hbm.at[grp], col_v)
        pltpu.sync_copy(row_hbm.at[grp], row_v)
        pltpu.sync_copy(val_hbm.at[grp], val_v)
        pltpu.async_copy(x_hbm.at[col_v], rows_v, sem).wait()
        for ch in range(G // 16):
            off = ch * 16
            v16 = val_v[pl.ds(off, 16)]
            if use_dis:
                r16 = row_v[pl.ds(off, 16)]
                c16 = col_v[pl.ds(off, 16)]
                v16 = v16 * plsc.load_gather(dis_v, [r16]) \
                          * plsc.load_gather(dis_v, [c16])
            iota16 = lax.iota(jnp.int32, 16)
            for r in range(16):
                splat = jnp.broadcast_to(
                    jnp.sum(jnp.where(iota16 == r, v16, 0.0)), (16,))
                row_i = off + r
                for j in range(D // 16):
                    rows_v[row_i, pl.ds(j * 16, 16)] = \
                        rows_v[row_i, pl.ds(j * 16, 16)] * splat
        pltpu.sync_copy(rows_v, acc.at[row_v], add=True)
        return 0

    lax.fori_loop(0, GP_TEC, grp_body, 0)
    plsc.subcore_barrier()
    base = s * RPS
    pltpu.sync_copy(acc.at[pl.ds(base, RPS)], out_hbm.at[c, pl.ds(base, RPS)])


def _make_spmm(use_dis):
    mesh = plsc.VectorSubcoreMesh(core_axis_name="c", subcore_axis_name="s",
                                  num_cores=NC, num_subcores=NS)
    scratch = [
        pltpu.VMEM((G,), jnp.int32),
        pltpu.VMEM((G,), jnp.int32),
        pltpu.VMEM((G,), jnp.float32),
        pltpu.VMEM((G, D), jnp.float32),
        pltpu.VMEM((N,), jnp.float32),
        pltpu.VMEM_SHARED((N, D), jnp.float32),
        pltpu.SemaphoreType.DMA,
    ]
    return pl.kernel(
        functools.partial(_spmm_body, use_dis),
        out_type=jax.ShapeDtypeStruct((NC, N, D), jnp.float32),
        mesh=mesh,
        scratch_types=scratch,
        compiler_params=pltpu.CompilerParams(use_tc_tiling_on_sc=False,
                                             needs_layout_passes=False),
        name="spmm_dis" if use_dis else "spmm_plain",
    )


def _deg_body(row_hbm, val_hbm, out_hbm, row_v, val_v, buf_v, acc, sem):
    c = lax.axis_index("c")
    s = lax.axis_index("s")
    _zero_rows(buf_v, G, 16)
    _zero_acc_slice(buf_v, acc, s * RPS, 16)
    plsc.subcore_barrier()

    g0 = (c * NS + s) * GP_TEC
    iota = lax.iota(jnp.int32, 16)
    zcol = jnp.zeros((16,), jnp.int32)

    def grp_body(g, _):
        grp = g0 + g
        pltpu.sync_copy(row_hbm.at[grp], row_v)
        pltpu.sync_copy(val_hbm.at[grp], val_v)
        for ch in range(G // 16):
            off = ch * 16
            v16 = val_v[pl.ds(off, 16)]
            plsc.store_scatter(buf_v, [iota + off, zcol], v16)
        pltpu.sync_copy(buf_v, acc.at[row_v], add=True)
        return 0

    lax.fori_loop(0, GP_TEC, grp_body, 0)
    plsc.subcore_barrier()
    base = s * RPS
    pltpu.sync_copy(acc.at[pl.ds(base, RPS)], out_hbm.at[c, pl.ds(base, RPS)])


def _make_deg():
    mesh = plsc.VectorSubcoreMesh(core_axis_name="c", subcore_axis_name="s",
                                  num_cores=NC, num_subcores=NS)
    scratch = [
        pltpu.VMEM((G,), jnp.int32),
        pltpu.VMEM((G,), jnp.float32),
        pltpu.VMEM((G, 16), jnp.float32),
        pltpu.VMEM_SHARED((N, 16), jnp.float32),
        pltpu.SemaphoreType.DMA,
    ]
    return pl.kernel(
        _deg_body,
        out_type=jax.ShapeDtypeStruct((NC, N, 16), jnp.float32),
        mesh=mesh,
        scratch_types=scratch,
        compiler_params=pltpu.CompilerParams(use_tc_tiling_on_sc=False,
                                             needs_layout_passes=False),
        name="deg_seg_sum",
    )


@functools.lru_cache(maxsize=None)
def _get_kernels():
    return _make_spmm(True), _make_spmm(False), _make_deg()


def _pack_edges(indices, values):
    pad = EPAD - E
    row = jnp.concatenate([indices[0].astype(jnp.int32),
                           jnp.zeros((pad,), jnp.int32)]).reshape(NGRP, G)
    col = jnp.concatenate([indices[1].astype(jnp.int32),
                           jnp.zeros((pad,), jnp.int32)]).reshape(NGRP, G)
    val = jnp.concatenate([values,
                           jnp.zeros((pad,), jnp.float32)]).reshape(NGRP, G)
    return row, col, val


def kernel(x, shuf, adj_indices, adj_values, neighbor_indices,
           neighbor_values, temp):
    _spmm_dis, _spmm_plain, _deg_kernel = _get_kernels()
    _dummy_dis = jnp.zeros((N,), jnp.float32)
    a_row, a_col, a_val = _pack_edges(adj_indices, adj_values)
    n_row, n_col, n_val = _pack_edges(neighbor_indices, neighbor_values)

    degp = _deg_kernel(a_row, a_val)
    deg = degp[0, :, 0] + degp[1, :, 0]
    dis = jnp.where(deg > 0, lax.rsqrt(jnp.where(deg > 0, deg, 1.0)), 0.0)

    u1p = _spmm_dis(a_col, a_row, a_val, x, dis)
    u1 = u1p[0] + u1p[1]
    u2p = _spmm_dis(a_col, a_row, a_val, u1, dis)
    u2 = u2p[0] + u2p[1]

    T = jax.nn.relu(temp)
    a = T[0]
    b = T[1] - T[0]
    c4 = (T[0] + T[2] - 2.0 * T[1]) / 4.0
    out = (a + b + c4) * x - (b + 2.0 * c4) * u1 + c4 * u2

    tp = _spmm_plain(n_col, n_row, n_val, out, _dummy_dis)
    tpos = tp[0] + tp[1]
    zp = _spmm_plain(n_col, n_row, n_val, tpos, _dummy_dis)
    z_pos = zp[0] + zp[1]

    out_shuf = out[shuf, :]
    tn = _spmm_plain(n_col, n_row, n_val, out_shuf, _dummy_dis)
    tneg = tn[0] + tn[1]
    zn = _spmm_plain(n_col, n_row, n_val, tneg, _dummy_dis)
    z_neg = zn[0] + zn[1]

    return out, z_pos, z_neg
